# Initial kernel scaffold; baseline (speedup 1.0000x reference)
import jax
import jax.numpy as jnp
from jax.experimental import pallas as pl


def kernel(edge_index, edge_vals, inputs, weight, bias):
    raise NotImplementedError("write your pallas kernel here")


# speedup vs baseline: 1.6950x; 1.6950x over previous
import functools

import jax
import jax.numpy as jnp
from jax import lax
from jax.experimental import pallas as pl
from jax.experimental.pallas import tpu as pltpu
from jax.experimental.pallas import tpu_sc as plsc

B, V, E, FIN, K, FOUT = 4, 10000, 160000, 256, 5, 256

NC, NS, L = 2, 16, 16

W = 128
NCH = (B * FIN) // W
CPC = NCH // NC
EPT = E // NS
EB = 80
NBLK = EPT // EB
RPT = V // NS
RB = 25
NRB = RPT // RB


def _spmm_body(x_hbm, col_hbm, row_hbm, val_hbm, prev_hbm, y_hbm,
               acc, colv, rowv, valv, gidxv, rows_v, zbuf, wbv, pbv,
               gsem, ssem, *, has_prev):
  cid = lax.axis_index("c")
  sid = lax.axis_index("s")

  @pl.loop(0, RB)
  def _zero_init(i):
    for j in range(W // L):
      zbuf[i, pl.ds(j * L, L)] = jnp.zeros((L,), jnp.float32)

  @pl.loop(0, CPC)
  def _chunk_loop(ci):
    chunk = cid * CPC + ci
    @pl.loop(0, NRB)
    def _zero(wb):
      r0 = sid * RPT + wb * RB
      pltpu.sync_copy(zbuf, acc.at[pl.ds(r0, RB)])

    plsc.subcore_barrier()

    ebase = sid * EPT

    @pl.loop(0, NBLK)
    def _edges(blk):
      off = ebase + blk * EB
      pltpu.sync_copy(col_hbm.at[pl.ds(off, EB)], colv)
      pltpu.sync_copy(row_hbm.at[pl.ds(off, EB)], rowv)
      pltpu.sync_copy(val_hbm.at[pl.ds(off, EB)], valv)
      cbase = chunk * V
      for j in range(EB // L):
        gidxv[pl.ds(j * L, L)] = colv[pl.ds(j * L, L)] + cbase
      pltpu.async_copy(x_hbm.at[gidxv], rows_v, gsem).wait()

      @pl.loop(0, EB // L)
      def _scale(g):
        v16 = valv[pl.ds(g * L, L)]
        if has_prev:
          v16 = v16 * 2.0
        for i in range(L):
          val = v16[i]
          e = g * L + i
          for j in range(W // L):
            sl = pl.ds(j * L, L)
            rows_v[e, sl] = rows_v[e, sl] * val

      pltpu.async_copy(rows_v, acc.at[rowv], ssem, add=True).wait()

    plsc.subcore_barrier()

    @pl.loop(0, NRB)
    def _wb(wb):
      r0 = sid * RPT + wb * RB
      pltpu.sync_copy(acc.at[pl.ds(r0, RB)], wbv)
      if has_prev:
        pltpu.sync_copy(prev_hbm.at[pl.ds(chunk * V + r0, RB)], pbv)

        @pl.loop(0, RB)
        def _sub(i):
          for j in range(W // L):
            sl = pl.ds(j * L, L)
            wbv[i, sl] = wbv[i, sl] - pbv[i, sl]

      pltpu.sync_copy(wbv, y_hbm.at[pl.ds(chunk * V + r0, RB)])

    plsc.subcore_barrier()


def _make_spmm(has_prev):
  mesh = plsc.VectorSubcoreMesh(core_axis_name="c", subcore_axis_name="s")
  return pl.kernel(
      functools.partial(_spmm_body, has_prev=has_prev),
      out_type=jax.ShapeDtypeStruct((NCH * V, W), jnp.float32),
      mesh=mesh,
      scratch_types=[
          pltpu.VMEM_SHARED((V, W), jnp.float32),
          pltpu.VMEM((EB,), jnp.int32),
          pltpu.VMEM((EB,), jnp.int32),
          pltpu.VMEM((EB,), jnp.float32),
          pltpu.VMEM((EB,), jnp.int32),
          pltpu.VMEM((EB, W), jnp.float32),
          pltpu.VMEM((RB, W), jnp.float32),
          pltpu.VMEM((RB, W), jnp.float32),
          pltpu.VMEM((RB, W), jnp.float32),
          pltpu.SemaphoreType.DMA,
          pltpu.SemaphoreType.DMA,
      ],
      compiler_params=pltpu.CompilerParams(use_tc_tiling_on_sc=False),
      name="cheb_spmm",
  )


_spmm_first = _make_spmm(False)
_spmm_cheb = _make_spmm(True)


def _matmul_kernel(x0, x1, x2, x3, x4, wt, bias, out):
  acc = jnp.zeros((out.shape[1], FOUT), jnp.float32)
  for k, xr in enumerate((x0, x1, x2, x3, x4)):
    for h in range(2):
      acc += jnp.dot(xr[h], wt[k, h], preferred_element_type=jnp.float32)
  out[0] = acc + bias[0]


VB = 1000


def _matmul(xs, wt, bias):
  grid = (B, V // VB)
  x_spec = pl.BlockSpec((2, VB, W), lambda b, vb: (b, vb, 0))
  return pl.pallas_call(
      _matmul_kernel,
      grid=grid,
      in_specs=[x_spec] * K + [
          pl.BlockSpec((K, 2, W, FOUT), lambda b, vb: (0, 0, 0, 0)),
          pl.BlockSpec((1, FOUT), lambda b, vb: (0, 0)),
      ],
      out_specs=pl.BlockSpec((1, VB, FOUT), lambda b, vb: (b, vb, 0)),
      out_shape=jax.ShapeDtypeStruct((B, V, FOUT), jnp.float32),
  )(*xs, wt, bias)


def kernel(edge_index, edge_vals, inputs, weight, bias):
  row = edge_index[0]
  col = edge_index[1]
  x0 = inputs.reshape(B, V, 2, W).transpose(0, 2, 1, 3).reshape(NCH * V, W)
  x1 = _spmm_first(x0, col, row, edge_vals, x0)
  x2 = _spmm_cheb(x1, col, row, edge_vals, x0)
  x3 = _spmm_cheb(x2, col, row, edge_vals, x1)
  x4 = _spmm_cheb(x3, col, row, edge_vals, x2)
  wt = weight.transpose(1, 0, 2).reshape(K, 2, W, FOUT)
  xs = [x.reshape(NCH, V, W) for x in (x0, x1, x2, x3, x4)]
  return _matmul(xs, wt, bias.reshape(1, FOUT))


# speedup vs baseline: 4.2627x; 2.5149x over previous
import functools

import jax
import jax.numpy as jnp
from jax import lax
from jax.experimental import pallas as pl
from jax.experimental.pallas import tpu as pltpu
from jax.experimental.pallas import tpu_sc as plsc

B, V, E, FIN, K, FOUT = 4, 10000, 160000, 256, 5, 256

NC, NS, L = 2, 16, 16

W = 128
NCH = (B * FIN) // W
CPC = NCH // NC
EPT = E // NS
EB = 80
NBLK = EPT // EB
RPT = V // NS
RB = 25
NRB = RPT // RB


def _spmm_body(x_hbm, col_hbm, row_hbm, val_hbm, prev_hbm, y_hbm,
               acc,
               col0, row0, val0, gidx0, srow0, rows0,
               col1, row1, val1, gidx1, srow1, rows1,
               wbv0, pbv0, wbv1, pbv1,
               isem0, gsem0, ssem0, isem1, gsem1, ssem1,
               wsem, lsem0, stsem0, lsem1, stsem1,
               *, has_prev):
  cid = lax.axis_index("c")
  sid = lax.axis_index("s")
  ebase = sid * EPT

  EBUF0 = (col0, row0, val0, gidx0, srow0, rows0, isem0, gsem0, ssem0)
  EBUF1 = (col1, row1, val1, gidx1, srow1, rows1, isem1, gsem1, ssem1)
  WBUF0 = (wbv0, pbv0, lsem0, stsem0)
  WBUF1 = (wbv1, pbv1, lsem1, stsem1)

  def idx_issue(blk, b):
    off = ebase + blk * EB
    pltpu.async_copy(col_hbm.at[pl.ds(off, EB)], b[0], b[6])
    pltpu.async_copy(row_hbm.at[pl.ds(off, EB)], b[1], b[6])
    pltpu.async_copy(val_hbm.at[pl.ds(off, EB)], b[2], b[6])

  def idx_wait(b):
    pltpu.make_async_copy(col_hbm.at[pl.ds(ebase, EB)], b[0], b[6]).wait()
    pltpu.make_async_copy(row_hbm.at[pl.ds(ebase, EB)], b[1], b[6]).wait()
    pltpu.make_async_copy(val_hbm.at[pl.ds(ebase, EB)], b[2], b[6]).wait()

  def gather_issue(b, cbase):
    for j in range(EB // L):
      sl = pl.ds(j * L, L)
      b[3][sl] = b[0][sl] + cbase
    pltpu.async_copy(x_hbm.at[b[3]], b[5], b[7])

  def gather_wait(b):
    pltpu.make_async_copy(x_hbm.at[b[3]], b[5], b[7]).wait()

  def scatter_issue(b):
    for j in range(EB // L):
      sl = pl.ds(j * L, L)
      b[4][sl] = b[1][sl]
    pltpu.async_copy(b[5], acc.at[b[4]], b[8], add=True)

  def scatter_wait(b):
    pltpu.make_async_copy(b[5], acc.at[b[4]], b[8]).wait()

  def scale(b):
    @pl.loop(0, EB // L)
    def _sg(g):
      v16 = b[2][pl.ds(g * L, L)]
      if has_prev:
        v16 = v16 * 2.0
      for i in range(L):
        val = v16[i]
        e = g * L + i
        for j in range(W // L):
          sl = pl.ds(j * L, L)
          b[5][e, sl] = b[5][e, sl] * val

  def half(blk, A, Bb, cbase):
    @pl.when(blk + 1 < NBLK)
    def _pf():
      idx_wait(Bb)
      @pl.when(blk >= 1)
      def _dr():
        scatter_wait(Bb)
      gather_issue(Bb, cbase)
    gather_wait(A)
    scale(A)
    scatter_issue(A)
    @pl.when(blk + 2 < NBLK)
    def _nidx():
      idx_issue(blk + 2, A)

  def wb_issue(wb, b, cbase):
    r0 = sid * RPT + wb * RB
    pltpu.async_copy(acc.at[pl.ds(r0, RB)], b[0], b[2])
    if has_prev:
      pltpu.async_copy(prev_hbm.at[pl.ds(cbase + r0, RB)], b[1], b[2])

  def wb_wait(b, cbase):
    pltpu.make_async_copy(acc.at[pl.ds(sid * RPT, RB)], b[0], b[2]).wait()
    if has_prev:
      pltpu.make_async_copy(
          prev_hbm.at[pl.ds(cbase, RB)], b[1], b[2]).wait()

  def st_issue(wb, b, cbase):
    r0 = sid * RPT + wb * RB
    pltpu.async_copy(b[0], y_hbm.at[pl.ds(cbase + r0, RB)], b[3])

  def st_wait(b, cbase):
    pltpu.make_async_copy(b[0], y_hbm.at[pl.ds(cbase, RB)], b[3]).wait()

  def wb_half(wb, A, Bb, cbase):
    @pl.when(wb + 1 < NRB)
    def _pf():
      @pl.when(wb >= 1)
      def _dr():
        st_wait(Bb, cbase)
      wb_issue(wb + 1, Bb, cbase)
    wb_wait(A, cbase)
    if has_prev:
      @pl.loop(0, RB)
      def _sub(i):
        for j in range(W // L):
          sl = pl.ds(j * L, L)
          A[0][i, sl] = A[0][i, sl] - A[1][i, sl]
    st_issue(wb, A, cbase)

  @pl.loop(0, CPC)
  def _chunk_loop(ci):
    chunk = cid * CPC + ci
    cbase = chunk * V

    @pl.loop(0, RB)
    def _zfill(i):
      for j in range(W // L):
        wbv0[i, pl.ds(j * L, L)] = jnp.zeros((L,), jnp.float32)

    @pl.loop(0, NRB)
    def _zissue(wb):
      pltpu.sync_copy(wbv0, acc.at[pl.ds(sid * RPT + wb * RB, RB)])

    plsc.subcore_barrier()

    idx_issue(0, EBUF0)
    idx_issue(1, EBUF1)
    idx_wait(EBUF0)
    gather_issue(EBUF0, cbase)

    @pl.loop(0, (NBLK - 1) // 2)
    def _pairs(i):
      half(2 * i, EBUF0, EBUF1, cbase)
      half(2 * i + 1, EBUF1, EBUF0, cbase)

    half(NBLK - 1, EBUF0, EBUF1, cbase)
    scatter_wait(EBUF1)
    scatter_wait(EBUF0)

    plsc.subcore_barrier()

    @pl.loop(0, NRB)
    def _wb(wb):
      r0 = sid * RPT + wb * RB
      pltpu.sync_copy(acc.at[pl.ds(r0, RB)], wbv0)
      if has_prev:
        pltpu.sync_copy(prev_hbm.at[pl.ds(cbase + r0, RB)], pbv0)

        @pl.loop(0, RB)
        def _sub(i):
          for j in range(W // L):
            sl = pl.ds(j * L, L)
            wbv0[i, sl] = wbv0[i, sl] - pbv0[i, sl]

      pltpu.sync_copy(wbv0, y_hbm.at[pl.ds(cbase + r0, RB)])

    plsc.subcore_barrier()


def _make_spmm(has_prev):
  mesh = plsc.VectorSubcoreMesh(core_axis_name="c", subcore_axis_name="s")
  return pl.kernel(
      functools.partial(_spmm_body, has_prev=has_prev),
      out_type=jax.ShapeDtypeStruct((NCH * V, W), jnp.float32),
      mesh=mesh,
      scratch_types=[
          pltpu.VMEM_SHARED((V, W), jnp.float32),
          pltpu.VMEM((EB,), jnp.int32),
          pltpu.VMEM((EB,), jnp.int32),
          pltpu.VMEM((EB,), jnp.float32),
          pltpu.VMEM((EB,), jnp.int32),
          pltpu.VMEM((EB,), jnp.int32),
          pltpu.VMEM((EB, W), jnp.float32),
          pltpu.VMEM((EB,), jnp.int32),
          pltpu.VMEM((EB,), jnp.int32),
          pltpu.VMEM((EB,), jnp.float32),
          pltpu.VMEM((EB,), jnp.int32),
          pltpu.VMEM((EB,), jnp.int32),
          pltpu.VMEM((EB, W), jnp.float32),
          pltpu.VMEM((RB, W), jnp.float32),
          pltpu.VMEM((RB, W), jnp.float32),
          pltpu.VMEM((RB, W), jnp.float32),
          pltpu.VMEM((RB, W), jnp.float32),
          pltpu.SemaphoreType.DMA,
          pltpu.SemaphoreType.DMA,
          pltpu.SemaphoreType.DMA,
          pltpu.SemaphoreType.DMA,
          pltpu.SemaphoreType.DMA,
          pltpu.SemaphoreType.DMA,
          pltpu.SemaphoreType.DMA,
          pltpu.SemaphoreType.DMA,
          pltpu.SemaphoreType.DMA,
          pltpu.SemaphoreType.DMA,
          pltpu.SemaphoreType.DMA,
      ],
      compiler_params=pltpu.CompilerParams(use_tc_tiling_on_sc=False),
      name="cheb_spmm",
  )


_spmm_first = _make_spmm(False)
_spmm_cheb = _make_spmm(True)


def _matmul_kernel(x0, x1, x2, x3, x4, wt, bias, out):
  acc = jnp.zeros((out.shape[1], FOUT), jnp.float32)
  for k, xr in enumerate((x0, x1, x2, x3, x4)):
    for h in range(2):
      acc += jnp.dot(xr[h], wt[k, h], preferred_element_type=jnp.float32)
  out[0] = acc + bias[0]


VB = 1000


def _matmul(xs, wt, bias):
  grid = (B, V // VB)
  x_spec = pl.BlockSpec((2, VB, W), lambda b, vb: (b, vb, 0))
  return pl.pallas_call(
      _matmul_kernel,
      grid=grid,
      in_specs=[x_spec] * K + [
          pl.BlockSpec((K, 2, W, FOUT), lambda b, vb: (0, 0, 0, 0)),
          pl.BlockSpec((1, FOUT), lambda b, vb: (0, 0)),
      ],
      out_specs=pl.BlockSpec((1, VB, FOUT), lambda b, vb: (b, vb, 0)),
      out_shape=jax.ShapeDtypeStruct((B, V, FOUT), jnp.float32),
  )(*xs, wt, bias)


def kernel(edge_index, edge_vals, inputs, weight, bias):
  row = edge_index[0]
  col = edge_index[1]
  x0 = inputs.reshape(B, V, 2, W).transpose(0, 2, 1, 3).reshape(NCH * V, W)
  x1 = _spmm_first(x0, col, row, edge_vals, x0)
  x2 = _spmm_cheb(x1, col, row, edge_vals, x0)
  x3 = _spmm_cheb(x2, col, row, edge_vals, x1)
  x4 = _spmm_cheb(x3, col, row, edge_vals, x2)
  wt = weight.transpose(1, 0, 2).reshape(K, 2, W, FOUT)
  xs = [x.reshape(NCH, V, W) for x in (x0, x1, x2, x3, x4)]
  return _matmul(xs, wt, bias.reshape(1, FOUT))


# speedup vs baseline: 4.5120x; 1.0585x over previous
import functools

import jax
import jax.numpy as jnp
from jax import lax
from jax.experimental import pallas as pl
from jax.experimental.pallas import tpu as pltpu
from jax.experimental.pallas import tpu_sc as plsc

B, V, E, FIN, K, FOUT = 4, 10000, 160000, 256, 5, 256

NC, NS, L = 2, 16, 16

W = 128
NCH = (B * FIN) // W
CPC = NCH // NC
EPT = E // NS
EB = 80
NBLK = EPT // EB
RPT = V // NS
RB = 25
NRB = RPT // RB


def _spmm_body(x_hbm, col_hbm, row_hbm, val_hbm, prev_hbm, y_hbm,
               acc,
               col0, row0, val0, gidx0, srow0, rows0,
               col1, row1, val1, gidx1, srow1, rows1,
               wbv0, pbv0, wbv1, pbv1,
               isem0, gsem0, ssem0, isem1, gsem1, ssem1,
               wsem, lsem0, stsem0, lsem1, stsem1,
               *, has_prev):
  cid = lax.axis_index("c")
  sid = lax.axis_index("s")
  ebase = sid * EPT

  EBUF0 = (col0, row0, val0, gidx0, srow0, rows0, isem0, gsem0, ssem0)
  EBUF1 = (col1, row1, val1, gidx1, srow1, rows1, isem1, gsem1, ssem1)
  WBUF0 = (wbv0, pbv0, lsem0, stsem0)
  WBUF1 = (wbv1, pbv1, lsem1, stsem1)

  def idx_issue(blk, b):
    off = ebase + blk * EB
    pltpu.async_copy(col_hbm.at[pl.ds(off, EB)], b[0], b[6])
    pltpu.async_copy(row_hbm.at[pl.ds(off, EB)], b[1], b[6])
    pltpu.async_copy(val_hbm.at[pl.ds(off, EB)], b[2], b[6])

  def idx_wait(b):
    pltpu.make_async_copy(col_hbm.at[pl.ds(ebase, EB)], b[0], b[6]).wait()
    pltpu.make_async_copy(row_hbm.at[pl.ds(ebase, EB)], b[1], b[6]).wait()
    pltpu.make_async_copy(val_hbm.at[pl.ds(ebase, EB)], b[2], b[6]).wait()

  def gather_issue(b, cbase):
    for j in range(EB // L):
      sl = pl.ds(j * L, L)
      b[3][sl] = b[0][sl] + cbase
    pltpu.async_copy(x_hbm.at[b[3]], b[5], b[7])

  def gather_wait(b):
    pltpu.make_async_copy(x_hbm.at[b[3]], b[5], b[7]).wait()

  def scatter_issue(b):
    for j in range(EB // L):
      sl = pl.ds(j * L, L)
      b[4][sl] = b[1][sl]
    pltpu.async_copy(b[5], acc.at[b[4]], b[8], add=True)

  def scatter_wait(b):
    pltpu.make_async_copy(b[5], acc.at[b[4]], b[8]).wait()

  def scale(b):
    @pl.loop(0, EB // L)
    def _sg(g):
      v16 = b[2][pl.ds(g * L, L)]
      if has_prev:
        v16 = v16 * 2.0
      for i in range(L):
        val = v16[i]
        e = g * L + i
        for j in range(W // L):
          sl = pl.ds(j * L, L)
          b[5][e, sl] = b[5][e, sl] * val

  def half(blk, A, Bb, cbase):
    @pl.when(blk + 1 < NBLK)
    def _pf():
      idx_wait(Bb)
      @pl.when(blk >= 1)
      def _dr():
        scatter_wait(Bb)
      gather_issue(Bb, cbase)
    gather_wait(A)
    scale(A)
    scatter_issue(A)
    @pl.when(blk + 2 < NBLK)
    def _nidx():
      idx_issue(blk + 2, A)

  @pl.loop(0, CPC)
  def _chunk_loop(ci):
    chunk = cid * CPC + ci
    cbase = chunk * V

    @pl.loop(0, RB)
    def _zfill(i):
      for j in range(W // L):
        wbv0[i, pl.ds(j * L, L)] = jnp.zeros((L,), jnp.float32)

    @pl.loop(0, NRB)
    def _zissue(wb):
      pltpu.sync_copy(wbv0, acc.at[pl.ds(sid * RPT + wb * RB, RB)])

    plsc.subcore_barrier()

    idx_issue(0, EBUF0)
    idx_issue(1, EBUF1)
    idx_wait(EBUF0)
    gather_issue(EBUF0, cbase)

    @pl.loop(0, (NBLK - 1) // 2)
    def _pairs(i):
      half(2 * i, EBUF0, EBUF1, cbase)
      half(2 * i + 1, EBUF1, EBUF0, cbase)

    half(NBLK - 1, EBUF0, EBUF1, cbase)
    scatter_wait(EBUF1)
    scatter_wait(EBUF0)

    plsc.subcore_barrier()

    @pl.loop(0, NRB // 2)
    def _wb(i):
      wb = 2 * i
      r0 = sid * RPT + wb * RB
      r1 = r0 + RB
      if has_prev:
        dpa = pltpu.async_copy(prev_hbm.at[pl.ds(cbase + r0, RB)], pbv0, lsem0)
        dpb = pltpu.async_copy(prev_hbm.at[pl.ds(cbase + r1, RB)], pbv1, lsem1)
      pltpu.sync_copy(acc.at[pl.ds(r0, RB)], wbv0)
      if has_prev:
        dpa.wait()

        @pl.loop(0, RB)
        def _sub0(r):
          for j in range(W // L):
            sl = pl.ds(j * L, L)
            wbv0[r, sl] = wbv0[r, sl] - pbv0[r, sl]

      sta = pltpu.async_copy(wbv0, y_hbm.at[pl.ds(cbase + r0, RB)], stsem0)
      pltpu.sync_copy(acc.at[pl.ds(r1, RB)], wbv1)
      if has_prev:
        dpb.wait()

        @pl.loop(0, RB)
        def _sub1(r):
          for j in range(W // L):
            sl = pl.ds(j * L, L)
            wbv1[r, sl] = wbv1[r, sl] - pbv1[r, sl]

      stb = pltpu.async_copy(wbv1, y_hbm.at[pl.ds(cbase + r1, RB)], stsem1)
      sta.wait()
      stb.wait()

    wbt = NRB - 1
    rt = sid * RPT + wbt * RB
    pltpu.sync_copy(acc.at[pl.ds(rt, RB)], wbv0)
    if has_prev:
      pltpu.sync_copy(prev_hbm.at[pl.ds(cbase + rt, RB)], pbv0)

      @pl.loop(0, RB)
      def _subt(r):
        for j in range(W // L):
          sl = pl.ds(j * L, L)
          wbv0[r, sl] = wbv0[r, sl] - pbv0[r, sl]

    pltpu.sync_copy(wbv0, y_hbm.at[pl.ds(cbase + rt, RB)])

    plsc.subcore_barrier()


def _make_spmm(has_prev):
  mesh = plsc.VectorSubcoreMesh(core_axis_name="c", subcore_axis_name="s")
  return pl.kernel(
      functools.partial(_spmm_body, has_prev=has_prev),
      out_type=jax.ShapeDtypeStruct((NCH * V, W), jnp.float32),
      mesh=mesh,
      scratch_types=[
          pltpu.VMEM_SHARED((V, W), jnp.float32),
          pltpu.VMEM((EB,), jnp.int32),
          pltpu.VMEM((EB,), jnp.int32),
          pltpu.VMEM((EB,), jnp.float32),
          pltpu.VMEM((EB,), jnp.int32),
          pltpu.VMEM((EB,), jnp.int32),
          pltpu.VMEM((EB, W), jnp.float32),
          pltpu.VMEM((EB,), jnp.int32),
          pltpu.VMEM((EB,), jnp.int32),
          pltpu.VMEM((EB,), jnp.float32),
          pltpu.VMEM((EB,), jnp.int32),
          pltpu.VMEM((EB,), jnp.int32),
          pltpu.VMEM((EB, W), jnp.float32),
          pltpu.VMEM((RB, W), jnp.float32),
          pltpu.VMEM((RB, W), jnp.float32),
          pltpu.VMEM((RB, W), jnp.float32),
          pltpu.VMEM((RB, W), jnp.float32),
          pltpu.SemaphoreType.DMA,
          pltpu.SemaphoreType.DMA,
          pltpu.SemaphoreType.DMA,
          pltpu.SemaphoreType.DMA,
          pltpu.SemaphoreType.DMA,
          pltpu.SemaphoreType.DMA,
          pltpu.SemaphoreType.DMA,
          pltpu.SemaphoreType.DMA,
          pltpu.SemaphoreType.DMA,
          pltpu.SemaphoreType.DMA,
          pltpu.SemaphoreType.DMA,
      ],
      compiler_params=pltpu.CompilerParams(use_tc_tiling_on_sc=False),
      name="cheb_spmm",
  )


_spmm_first = _make_spmm(False)
_spmm_cheb = _make_spmm(True)


def _matmul_kernel(x0, x1, x2, x3, x4, wt, bias, out):
  acc = jnp.zeros((out.shape[1], FOUT), jnp.float32)
  for k, xr in enumerate((x0, x1, x2, x3, x4)):
    for h in range(2):
      acc += jnp.dot(xr[h], wt[k, h], preferred_element_type=jnp.float32)
  out[0] = acc + bias[0]


VB = 1000


def _matmul(xs, wt, bias):
  grid = (B, V // VB)
  x_spec = pl.BlockSpec((2, VB, W), lambda b, vb: (b, vb, 0))
  return pl.pallas_call(
      _matmul_kernel,
      grid=grid,
      in_specs=[x_spec] * K + [
          pl.BlockSpec((K, 2, W, FOUT), lambda b, vb: (0, 0, 0, 0)),
          pl.BlockSpec((1, FOUT), lambda b, vb: (0, 0)),
      ],
      out_specs=pl.BlockSpec((1, VB, FOUT), lambda b, vb: (b, vb, 0)),
      out_shape=jax.ShapeDtypeStruct((B, V, FOUT), jnp.float32),
  )(*xs, wt, bias)


def kernel(edge_index, edge_vals, inputs, weight, bias):
  row = edge_index[0]
  col = edge_index[1]
  x0 = inputs.reshape(B, V, 2, W).transpose(0, 2, 1, 3).reshape(NCH * V, W)
  x1 = _spmm_first(x0, col, row, edge_vals, x0)
  x2 = _spmm_cheb(x1, col, row, edge_vals, x0)
  x3 = _spmm_cheb(x2, col, row, edge_vals, x1)
  x4 = _spmm_cheb(x3, col, row, edge_vals, x2)
  wt = weight.transpose(1, 0, 2).reshape(K, 2, W, FOUT)
  xs = [x.reshape(NCH, V, W) for x in (x0, x1, x2, x3, x4)]
  return _matmul(xs, wt, bias.reshape(1, FOUT))


# speedup vs baseline: 4.7205x; 1.0462x over previous
import functools

import jax
import jax.numpy as jnp
from jax import lax
from jax.experimental import pallas as pl
from jax.experimental.pallas import tpu as pltpu
from jax.experimental.pallas import tpu_sc as plsc

B, V, E, FIN, K, FOUT = 4, 10000, 160000, 256, 5, 256

NC, NS, L = 2, 16, 16

W = 128
NCH = (B * FIN) // W
CPC = NCH // NC
EPT = E // NS
EB = 80
NBLK = EPT // EB
RPT = V // NS
RB = 25
NRB = RPT // RB


def _spmm_body(x_hbm, col_hbm, row_hbm, val_hbm, prev_hbm, y_hbm,
               acc,
               col0, row0, val0, gidx0, srow0, rows0,
               col1, row1, val1, gidx1, srow1, rows1,
               col2, row2, val2, gidx2, srow2, rows2,
               wbv0, pbv0, wbv1, pbv1,
               isem0, gsem0, ssem0, isem1, gsem1, ssem1,
               isem2, gsem2, ssem2,
               wsem, lsem0, stsem0, lsem1, stsem1,
               *, has_prev):
  cid = lax.axis_index("c")
  sid = lax.axis_index("s")
  ebase = sid * EPT

  EBUF0 = (col0, row0, val0, gidx0, srow0, rows0, isem0, gsem0, ssem0)
  EBUF1 = (col1, row1, val1, gidx1, srow1, rows1, isem1, gsem1, ssem1)
  EBUF2 = (col2, row2, val2, gidx2, srow2, rows2, isem2, gsem2, ssem2)
  WBUF0 = (wbv0, pbv0, lsem0, stsem0)
  WBUF1 = (wbv1, pbv1, lsem1, stsem1)

  def idx_issue(blk, b):
    off = ebase + blk * EB
    pltpu.async_copy(col_hbm.at[pl.ds(off, EB)], b[0], b[6])
    pltpu.async_copy(row_hbm.at[pl.ds(off, EB)], b[1], b[6])
    pltpu.async_copy(val_hbm.at[pl.ds(off, EB)], b[2], b[6])

  def idx_wait(b):
    pltpu.make_async_copy(col_hbm.at[pl.ds(ebase, EB)], b[0], b[6]).wait()
    pltpu.make_async_copy(row_hbm.at[pl.ds(ebase, EB)], b[1], b[6]).wait()
    pltpu.make_async_copy(val_hbm.at[pl.ds(ebase, EB)], b[2], b[6]).wait()

  def gather_issue(b, cbase):
    for j in range(EB // L):
      sl = pl.ds(j * L, L)
      b[3][sl] = b[0][sl] + cbase
    pltpu.async_copy(x_hbm.at[b[3]], b[5], b[7])

  def gather_wait(b):
    pltpu.make_async_copy(x_hbm.at[b[3]], b[5], b[7]).wait()

  def scatter_issue(b):
    for j in range(EB // L):
      sl = pl.ds(j * L, L)
      b[4][sl] = b[1][sl]
    pltpu.async_copy(b[5], acc.at[b[4]], b[8], add=True)

  def scatter_wait(b):
    pltpu.make_async_copy(b[5], acc.at[b[4]], b[8]).wait()

  def scale(b):
    @pl.loop(0, EB // L)
    def _sg(g):
      v16 = b[2][pl.ds(g * L, L)]
      if has_prev:
        v16 = v16 * 2.0
      for i in range(L):
        val = v16[i]
        e = g * L + i
        for j in range(W // L):
          sl = pl.ds(j * L, L)
          b[5][e, sl] = b[5][e, sl] * val

  def step3(blk, A, Bp, cbase):
    @pl.when(blk + 2 < NBLK)
    def _pf():
      idx_wait(Bp)
      @pl.when(blk >= 1)
      def _dr():
        scatter_wait(Bp)
      gather_issue(Bp, cbase)
    gather_wait(A)
    scale(A)
    scatter_issue(A)
    @pl.when(blk + 3 < NBLK)
    def _nidx():
      idx_issue(blk + 3, A)

  @pl.loop(0, CPC)
  def _chunk_loop(ci):
    chunk = cid * CPC + ci
    cbase = chunk * V

    @pl.loop(0, RB)
    def _zfill(i):
      for j in range(W // L):
        wbv0[i, pl.ds(j * L, L)] = jnp.zeros((L,), jnp.float32)

    @pl.loop(0, NRB)
    def _zissue(wb):
      pltpu.sync_copy(wbv0, acc.at[pl.ds(sid * RPT + wb * RB, RB)])

    plsc.subcore_barrier()

    idx_issue(0, EBUF0)
    idx_issue(1, EBUF1)
    idx_issue(2, EBUF2)
    idx_wait(EBUF0)
    gather_issue(EBUF0, cbase)
    idx_wait(EBUF1)
    gather_issue(EBUF1, cbase)

    @pl.loop(0, NBLK // 3)
    def _trips(i):
      step3(3 * i, EBUF0, EBUF2, cbase)
      step3(3 * i + 1, EBUF1, EBUF0, cbase)
      step3(3 * i + 2, EBUF2, EBUF1, cbase)

    step3(NBLK - 2, EBUF0, EBUF2, cbase)
    step3(NBLK - 1, EBUF1, EBUF0, cbase)
    scatter_wait(EBUF2)
    scatter_wait(EBUF0)
    scatter_wait(EBUF1)

    plsc.subcore_barrier()

    @pl.loop(0, NRB // 2)
    def _wb(i):
      wb = 2 * i
      r0 = sid * RPT + wb * RB
      r1 = r0 + RB
      if has_prev:
        dpa = pltpu.async_copy(prev_hbm.at[pl.ds(cbase + r0, RB)], pbv0, lsem0)
        dpb = pltpu.async_copy(prev_hbm.at[pl.ds(cbase + r1, RB)], pbv1, lsem1)
      pltpu.sync_copy(acc.at[pl.ds(r0, RB)], wbv0)
      if has_prev:
        dpa.wait()

        @pl.loop(0, RB)
        def _sub0(r):
          for j in range(W // L):
            sl = pl.ds(j * L, L)
            wbv0[r, sl] = wbv0[r, sl] - pbv0[r, sl]

      sta = pltpu.async_copy(wbv0, y_hbm.at[pl.ds(cbase + r0, RB)], stsem0)
      pltpu.sync_copy(acc.at[pl.ds(r1, RB)], wbv1)
      if has_prev:
        dpb.wait()

        @pl.loop(0, RB)
        def _sub1(r):
          for j in range(W // L):
            sl = pl.ds(j * L, L)
            wbv1[r, sl] = wbv1[r, sl] - pbv1[r, sl]

      stb = pltpu.async_copy(wbv1, y_hbm.at[pl.ds(cbase + r1, RB)], stsem1)
      sta.wait()
      stb.wait()

    wbt = NRB - 1
    rt = sid * RPT + wbt * RB
    pltpu.sync_copy(acc.at[pl.ds(rt, RB)], wbv0)
    if has_prev:
      pltpu.sync_copy(prev_hbm.at[pl.ds(cbase + rt, RB)], pbv0)

      @pl.loop(0, RB)
      def _subt(r):
        for j in range(W // L):
          sl = pl.ds(j * L, L)
          wbv0[r, sl] = wbv0[r, sl] - pbv0[r, sl]

    pltpu.sync_copy(wbv0, y_hbm.at[pl.ds(cbase + rt, RB)])

    plsc.subcore_barrier()


def _make_spmm(has_prev):
  mesh = plsc.VectorSubcoreMesh(core_axis_name="c", subcore_axis_name="s")
  return pl.kernel(
      functools.partial(_spmm_body, has_prev=has_prev),
      out_type=jax.ShapeDtypeStruct((NCH * V, W), jnp.float32),
      mesh=mesh,
      scratch_types=[
          pltpu.VMEM_SHARED((V, W), jnp.float32),
          pltpu.VMEM((EB,), jnp.int32),
          pltpu.VMEM((EB,), jnp.int32),
          pltpu.VMEM((EB,), jnp.float32),
          pltpu.VMEM((EB,), jnp.int32),
          pltpu.VMEM((EB,), jnp.int32),
          pltpu.VMEM((EB, W), jnp.float32),
          pltpu.VMEM((EB,), jnp.int32),
          pltpu.VMEM((EB,), jnp.int32),
          pltpu.VMEM((EB,), jnp.float32),
          pltpu.VMEM((EB,), jnp.int32),
          pltpu.VMEM((EB,), jnp.int32),
          pltpu.VMEM((EB, W), jnp.float32),
          pltpu.VMEM((EB,), jnp.int32),
          pltpu.VMEM((EB,), jnp.int32),
          pltpu.VMEM((EB,), jnp.float32),
          pltpu.VMEM((EB,), jnp.int32),
          pltpu.VMEM((EB,), jnp.int32),
          pltpu.VMEM((EB, W), jnp.float32),
          pltpu.VMEM((RB, W), jnp.float32),
          pltpu.VMEM((RB, W), jnp.float32),
          pltpu.VMEM((RB, W), jnp.float32),
          pltpu.VMEM((RB, W), jnp.float32),
          pltpu.SemaphoreType.DMA,
          pltpu.SemaphoreType.DMA,
          pltpu.SemaphoreType.DMA,
          pltpu.SemaphoreType.DMA,
          pltpu.SemaphoreType.DMA,
          pltpu.SemaphoreType.DMA,
          pltpu.SemaphoreType.DMA,
          pltpu.SemaphoreType.DMA,
          pltpu.SemaphoreType.DMA,
          pltpu.SemaphoreType.DMA,
          pltpu.SemaphoreType.DMA,
          pltpu.SemaphoreType.DMA,
          pltpu.SemaphoreType.DMA,
          pltpu.SemaphoreType.DMA,
      ],
      compiler_params=pltpu.CompilerParams(use_tc_tiling_on_sc=False),
      name="cheb_spmm",
  )


_spmm_first = _make_spmm(False)
_spmm_cheb = _make_spmm(True)


def _matmul_kernel(x0, x1, x2, x3, x4, wt, bias, out):
  acc = jnp.zeros((out.shape[1], FOUT), jnp.float32)
  for k, xr in enumerate((x0, x1, x2, x3, x4)):
    for h in range(2):
      acc += jnp.dot(xr[h], wt[k, h], preferred_element_type=jnp.float32)
  out[0] = acc + bias[0]


VB = 1000


def _matmul(xs, wt, bias):
  grid = (B, V // VB)
  x_spec = pl.BlockSpec((2, VB, W), lambda b, vb: (b, vb, 0))
  return pl.pallas_call(
      _matmul_kernel,
      grid=grid,
      in_specs=[x_spec] * K + [
          pl.BlockSpec((K, 2, W, FOUT), lambda b, vb: (0, 0, 0, 0)),
          pl.BlockSpec((1, FOUT), lambda b, vb: (0, 0)),
      ],
      out_specs=pl.BlockSpec((1, VB, FOUT), lambda b, vb: (b, vb, 0)),
      out_shape=jax.ShapeDtypeStruct((B, V, FOUT), jnp.float32),
  )(*xs, wt, bias)


def kernel(edge_index, edge_vals, inputs, weight, bias):
  row = edge_index[0]
  col = edge_index[1]
  x0 = inputs.reshape(B, V, 2, W).transpose(0, 2, 1, 3).reshape(NCH * V, W)
  x1 = _spmm_first(x0, col, row, edge_vals, x0)
  x2 = _spmm_cheb(x1, col, row, edge_vals, x0)
  x3 = _spmm_cheb(x2, col, row, edge_vals, x1)
  x4 = _spmm_cheb(x3, col, row, edge_vals, x2)
  wt = weight.transpose(1, 0, 2).reshape(K, 2, W, FOUT)
  xs = [x.reshape(NCH, V, W) for x in (x0, x1, x2, x3, x4)]
  return _matmul(xs, wt, bias.reshape(1, FOUT))
